# Initial kernel scaffold; baseline (speedup 1.0000x reference)
import jax
import jax.numpy as jnp
from jax.experimental import pallas as pl


def kernel(positions, atomic_numbers, edge_index, batch, type_embed, W_edge, W_msg, W_upd, W_out):
    raise NotImplementedError("write your pallas kernel here")


# speedup vs baseline: 25.2926x; 25.2926x over previous
import functools
import numpy as np
import jax
import jax.numpy as jnp
from jax import lax
from jax.experimental import pallas as pl
from jax.experimental.pallas import tpu as pltpu
from jax.experimental.pallas import tpu_sc as plsc

N = 10000
E = 320000
D = 128
NB = 8
NSYS = 8
RMAX = 5.0

NP_ = 10008
NPTC = 10240
NROW = 4 * NPTC
NW = 32
EPW = 10240
EP = NW * EPW
CHUNK = 1024
NBATCH = 8
GRP = 8
ROWS_PER_TILE = NROW // 16


def _sc_edge_kernel(posx_h, posy_h, posz_h, types_h, src_h, dst_h, zeros_h,
                    out_h, posx, posy, posz, types, srcb, dstb, rbfb, rowb,
                    racc):
    cid = lax.axis_index("c")
    sid = lax.axis_index("s")
    wid = sid * 2 + cid

    pltpu.sync_copy(zeros_h, racc.at[pl.ds(sid * ROWS_PER_TILE, ROWS_PER_TILE)])
    pltpu.sync_copy(posx_h, posx)
    pltpu.sync_copy(posy_h, posy)
    pltpu.sync_copy(posz_h, posz)
    pltpu.sync_copy(types_h, types)
    plsc.subcore_barrier()

    lane = lax.iota(jnp.int32, 16)
    pi_over_r = jnp.float32(np.pi / RMAX)
    two_over_pi = jnp.float32(2.0 / np.pi)
    half_pi = jnp.float32(np.pi / 2)
    rmaxf = jnp.float32(RMAX)

    base = wid * EPW

    def chunk_body(ci, _c):
        cbase = pl.multiple_of(base + ci * CHUNK, 8)
        pltpu.sync_copy(src_h.at[pl.ds(cbase, CHUNK)], srcb)
        pltpu.sync_copy(dst_h.at[pl.ds(cbase, CHUNK)], dstb)

        def batch_body(j, _b):
            def group(g8, _):
                g0 = j * 128 + g8 * 16
                sv = srcb[pl.ds(g0, 16)]
                dv = dstb[pl.ds(g0, 16)]
                xs = plsc.load_gather(posx, [sv])
                ys = plsc.load_gather(posy, [sv])
                zs = plsc.load_gather(posz, [sv])
                xd = plsc.load_gather(posx, [dv])
                yd = plsc.load_gather(posy, [dv])
                zd = plsc.load_gather(posz, [dv])
                tv = plsc.load_gather(types, [sv])

                dx = xd - xs
                dy = yd - ys
                dz = zd - zs
                d2 = dx * dx + dy * dy + dz * dz + jnp.float32(1e-12)
                yi = jnp.int32(0x5F3759DF) - lax.shift_right_arithmetic(
                    plsc.bitcast(d2, jnp.int32), 1)
                ry = plsc.bitcast(yi, jnp.float32)
                ry = ry * (1.5 - 0.5 * d2 * ry * ry)
                ry = ry * (1.5 - 0.5 * d2 * ry * ry)
                ry = ry * (1.5 - 0.5 * d2 * ry * ry)
                d = d2 * ry
                dc = jnp.minimum(d, rmaxf)
                theta = dc * pi_over_r
                q = (theta * two_over_pi + 0.5).astype(jnp.int32)
                r = theta - q.astype(jnp.float32) * half_pi
                r2 = r * r
                sr = r + r * r2 * (jnp.float32(-1.6666654611e-1) + r2 *
                                   (jnp.float32(8.3321608736e-3) + r2 *
                                    jnp.float32(-1.9515295891e-4)))
                cr = 1.0 - 0.5 * r2 + r2 * r2 * (
                    jnp.float32(4.166664568298827e-2) + r2 *
                    (jnp.float32(-1.388731625493765e-3) + r2 *
                     jnp.float32(2.443315711809948e-5)))
                q1 = q == 1
                q0 = q == 0
                s1 = jnp.where(q0, sr, jnp.where(q1, cr, -sr))
                c1 = jnp.where(q0, cr, jnp.where(q1, -sr, -cr))
                fc = (0.5 * (c1 + 1.0)) * jnp.where(
                    d < rmaxf, jnp.float32(1.0), jnp.float32(0.0))
                g = fc * ry
                twoc = c1 + c1

                ridx = lane + jnp.int32(g0)
                sk_1 = s1
                sk = twoc * s1
                plsc.store_scatter(rbfb, [ridx, jnp.zeros((16,), jnp.int32)], s1 * g)
                plsc.store_scatter(rbfb, [ridx, jnp.full((16,), 1, jnp.int32)], sk * g)
                for k in range(2, NB):
                    sk_1, sk = sk, twoc * sk - sk_1
                    plsc.store_scatter(rbfb, [ridx, jnp.full((16,), k, jnp.int32)], sk * g)

                rows_v = dv * 4 + tv
                rowb[j, pl.ds(g8 * 16, 16)] = rows_v
                return 0

            lax.fori_loop(0, GRP, group, 0)
            pltpu.sync_copy(rbfb.at[pl.ds(j * 128, 128)],
                            racc.at[rowb.at[j]], add=True)
            return 0

        lax.fori_loop(0, NBATCH, batch_body, 0)
        return 0

    lax.fori_loop(0, EPW // CHUNK, chunk_body, 0)

    plsc.subcore_barrier()
    pltpu.sync_copy(racc.at[pl.ds(sid * ROWS_PER_TILE, ROWS_PER_TILE)],
                    out_h.at[cid, pl.ds(sid * ROWS_PER_TILE, ROWS_PER_TILE)])


def _sc_edge_pass(posx, posy, posz, types, src, dst):
    zeros = jnp.zeros((ROWS_PER_TILE, NB), jnp.float32)
    mesh = plsc.VectorSubcoreMesh(core_axis_name="c", subcore_axis_name="s")
    k = pl.kernel(
        _sc_edge_kernel,
        out_type=jax.ShapeDtypeStruct((2, NROW, NB), jnp.float32),
        mesh=mesh,
        compiler_params=pltpu.CompilerParams(needs_layout_passes=False,
                                             use_tc_tiling_on_sc=False),
        scratch_types=[
            pltpu.VMEM((NP_,), jnp.float32),
            pltpu.VMEM((NP_,), jnp.float32),
            pltpu.VMEM((NP_,), jnp.float32),
            pltpu.VMEM((NP_,), jnp.int32),
            pltpu.VMEM((CHUNK,), jnp.int32),
            pltpu.VMEM((CHUNK,), jnp.int32),
            pltpu.VMEM((CHUNK, NB), jnp.float32),
            pltpu.VMEM((NBATCH, 128), jnp.int32),
            pltpu.VMEM_SHARED((NROW, NB), jnp.float32),
        ],
    )
    return k(posx, posy, posz, types, src, dst, zeros)


def _tc_tail_kernel(r2_ref, types_ref, batch_ref, te_ref, we_ref, wm_ref,
                    wu_ref, wo_ref, out_ref):
    i = pl.program_id(0)
    rblk = r2_ref[0] + r2_ref[1]
    hm4 = jnp.dot(te_ref[...], wm_ref[...], preferred_element_type=jnp.float32)
    we = we_ref[...]
    wflat = jnp.concatenate([we * hm4[t:t + 1, :] for t in range(4)], axis=0)
    agg = jnp.dot(rblk, wflat, preferred_element_type=jnp.float32)
    tv = types_ref[0]
    cols4 = lax.broadcasted_iota(jnp.int32, (tv.shape[0], 4), 1)
    oh = (tv == cols4).astype(jnp.float32)
    h = jnp.dot(oh, te_ref[...], preferred_element_type=jnp.float32)
    u = jnp.dot(agg, wu_ref[...], preferred_element_type=jnp.float32)
    h2 = u * (1.0 / (1.0 + jnp.exp(-u))) + h
    e_col = jnp.sum(h2 * wo_ref[...], axis=1, keepdims=True)
    bv = batch_ref[0]
    cols8 = lax.broadcasted_iota(jnp.int32, (bv.shape[0], NSYS), 1)
    msk = (bv == cols8).astype(jnp.float32)
    part = jnp.sum(e_col * msk, axis=0)[None, :]

    @pl.when(i == 0)
    def _():
        out_ref[...] = jnp.zeros_like(out_ref)

    out_ref[...] += part


def _tc_tail(r2, types3, batch3, type_embed, W_edge, W_msg, W_upd, W_outT):
    bn = 1024
    grid = (NPTC // bn,)
    return pl.pallas_call(
        _tc_tail_kernel,
        grid=grid,
        in_specs=[
            pl.BlockSpec((2, bn, 4 * NB), lambda i: (0, i, 0)),
            pl.BlockSpec((1, bn, 1), lambda i: (i, 0, 0)),
            pl.BlockSpec((1, bn, 1), lambda i: (i, 0, 0)),
            pl.BlockSpec((4, D), lambda i: (0, 0)),
            pl.BlockSpec((NB, D), lambda i: (0, 0)),
            pl.BlockSpec((D, D), lambda i: (0, 0)),
            pl.BlockSpec((D, D), lambda i: (0, 0)),
            pl.BlockSpec((1, D), lambda i: (0, 0)),
        ],
        out_specs=pl.BlockSpec((1, NSYS), lambda i: (0, 0)),
        out_shape=jax.ShapeDtypeStruct((1, NSYS), jnp.float32),
        compiler_params=pltpu.CompilerParams(
            dimension_semantics=("arbitrary",)),
    )(r2, types3, batch3, type_embed, W_edge, W_msg, W_upd, W_outT)


@jax.jit
def kernel(positions, atomic_numbers, edge_index, batch, type_embed, W_edge,
           W_msg, W_upd, W_out):
    z = atomic_numbers
    t = jnp.where(z == 1, 0, jnp.where(z == 6, 1, jnp.where(z == 7, 2, 3)))
    t = t.astype(jnp.int32)

    posx = jnp.zeros((NP_,), jnp.float32).at[:N].set(positions[:, 0])
    posy = jnp.zeros((NP_,), jnp.float32).at[:N].set(positions[:, 1])
    posz = jnp.zeros((NP_,), jnp.float32).at[:N].set(positions[:, 2])
    types = jnp.zeros((NP_,), jnp.int32).at[:N].set(t)

    src = jnp.zeros((EP,), jnp.int32).at[:E].set(edge_index[0])
    dst = jnp.full((EP,), N, jnp.int32).at[:E].set(edge_index[1])

    r2 = _sc_edge_pass(posx, posy, posz, types, src, dst)
    r2 = r2.reshape(2, NPTC, 4 * NB)

    types3 = jnp.zeros((NPTC,), jnp.int32).at[:N].set(t).reshape(
        NPTC // 1024, 1024, 1)
    batch3 = jnp.full((NPTC,), 127, jnp.int32).at[:N].set(batch).reshape(
        NPTC // 1024, 1024, 1)

    energy = _tc_tail(r2, types3, batch3, type_embed, W_edge, W_msg, W_upd,
                      W_out.reshape(1, D))
    return energy[0]
